# baseline (speedup 1.0000x reference)
import functools

import jax
import jax.numpy as jnp
from jax import lax
from jax.experimental import pallas as pl
from jax.experimental.pallas import tpu as pltpu
from jax.experimental.pallas import tpu_sc as plsc

NC, NS = 2, 16
NW = NC * NS

B, L, D = 4096, 200, 32
TD, TR, TC = D // 8, 8, 128
NU = L


@functools.partial(
    pl.kernel,
    out_type=(
        jax.ShapeDtypeStruct((L, TD, NW, TR, TC), jnp.float32),
        jax.ShapeDtypeStruct((TD, TR, TC), jnp.float32),
    ),
    mesh=plsc.VectorSubcoreMesh(core_axis_name="c", subcore_axis_name="s"),
    compiler_params=pltpu.CompilerParams(
        use_tc_tiling_on_sc=False, needs_layout_passes=False),
    scratch_types=[
        [pltpu.VMEM((1, TC), jnp.int32) for _ in range(4)],
        [pltpu.VMEM((TC, D), jnp.float32) for _ in range(4)],
        [pltpu.VMEM((D, TC), jnp.float32) for _ in range(2)],
        pltpu.VMEM((L, D), jnp.float32),
        [pltpu.SemaphoreType.DMA for _ in range(4)],
        [pltpu.SemaphoreType.DMA for _ in range(4)],
        [pltpu.SemaphoreType.DMA for _ in range(2)],
    ],
)
def _sc_embed(idxT_hbm, tab_hbm, pos_hbm, out5, dummy, idx_vs, rows_vs,
              trans_vs, pos_v, isems, gsems, ssems):
    wid = lax.axis_index("s") * NC + lax.axis_index("c")
    b0 = wid * TC
    pltpu.sync_copy(pos_hbm, pos_v)
    iota = lax.iota(jnp.int32, 16)

    def issue_idx(u, sl):
        pltpu.async_copy(idxT_hbm.at[pl.ds(u, 1), pl.ds(b0, TC)],
                         idx_vs[sl], isems[sl])

    def issue_gather(sl):
        pltpu.async_copy(tab_hbm.at[idx_vs[sl].at[0]], rows_vs[sl], gsems[sl])

    def wait_idx(sl):
        pltpu.make_async_copy(idxT_hbm.at[pl.ds(0, 1), pl.ds(0, TC)],
                              idx_vs[sl], isems[sl]).wait()

    def wait_gather(sl):
        pltpu.make_async_copy(tab_hbm.at[pl.ds(0, TC)],
                              rows_vs[sl], gsems[sl]).wait()

    def wait_store(sl):
        pltpu.make_async_copy(dummy, trans_vs[sl], ssems[sl]).wait()

    GRP = 8

    def unit(l, sl):
        rows_v = rows_vs[sl]
        trans_v = trans_vs[sl % 2]
        p0 = pos_v[l, pl.ds(0, 16)]
        p1 = pos_v[l, pl.ds(16, 16)]

        for c0 in range(0, TC, GRP):
            vals = []
            for c in range(c0, c0 + GRP):
                vals.append((rows_v[c, pl.ds(0, 16)] + p0,
                             rows_v[c, pl.ds(16, 16)] + p1))
            for c, (v0, v1) in zip(range(c0, c0 + GRP), vals):
                cv = jnp.full((16,), c, jnp.int32)
                plsc.store_scatter(trans_v, [iota, cv], v0)
                plsc.store_scatter(trans_v, [iota + 16, cv], v1)
        for td in range(TD):
            pltpu.async_copy(trans_v.at[pl.ds(td * TR, TR)],
                             out5.at[l, td, wid], ssems[sl % 2])

    pltpu.sync_copy(idxT_hbm.at[pl.ds(0, 1), pl.ds(b0, TC)], idx_vs[0])
    issue_gather(0)
    issue_idx(1, 1)
    issue_idx(2, 2)
    wait_idx(1)
    issue_gather(1)
    wait_idx(2)
    issue_gather(2)
    issue_idx(3, 3)

    def body(j, carry):
        for p in range(4):
            u = 4 * j + p
            @pl.when(u + 3 < NU)
            def _():
                wait_idx((p + 3) % 4)
                issue_gather((p + 3) % 4)

            @pl.when(u + 4 < NU)
            def _():
                issue_idx(u + 4, p)

            @pl.when(u >= 2)
            def _():
                wait_store(p % 2)

            wait_gather(p)
            unit(u, p)
        return carry

    lax.fori_loop(0, NU // 4, body, 0)
    wait_store(0)
    wait_store(1)


V = 1000000
TW = 16384
TG = -(-V // TW)


def _tc_relayout_body(x_ref, o_ref):
    y = x_ref[...].T.reshape(TW // 4, 4, D)
    o_ref[...] = jnp.concatenate([y[:, s, :] for s in range(4)], axis=-1)


_tc_relayout = pl.pallas_call(
    _tc_relayout_body,
    grid=(TG,),
    in_specs=[pl.BlockSpec((D, TW), lambda i: (0, i))],
    out_specs=pl.BlockSpec((TW // 4, 4 * D), lambda i: (i, 0)),
    out_shape=jax.ShapeDtypeStruct((V // 4, 4 * D), jnp.float32),
)


def kernel(inputs, token_table, pos_table):
    tab_lin = _tc_relayout(token_table.T).reshape(V, D)
    out5, _ = _sc_embed(inputs.T, tab_lin, pos_table)
    return out5.transpose(2, 4, 0, 1, 3).reshape(B, L, D)


# speedup vs baseline: 1.0156x; 1.0156x over previous
import functools

import jax
import jax.numpy as jnp
from jax import lax
from jax.experimental import pallas as pl
from jax.experimental.pallas import tpu as pltpu
from jax.experimental.pallas import tpu_sc as plsc

NC, NS = 2, 16
NW = NC * NS

B, L, D = 4096, 200, 32
TD, TR, TC = D // 8, 8, 128
NU = L


@functools.partial(
    pl.kernel,
    out_type=(
        jax.ShapeDtypeStruct((L, TD, NW, TR, TC), jnp.float32),
        jax.ShapeDtypeStruct((TD, TR, TC), jnp.float32),
    ),
    mesh=plsc.VectorSubcoreMesh(core_axis_name="c", subcore_axis_name="s"),
    compiler_params=pltpu.CompilerParams(
        use_tc_tiling_on_sc=False, needs_layout_passes=False),
    scratch_types=[
        [pltpu.VMEM((1, TC), jnp.int32) for _ in range(2)],
        [pltpu.VMEM((TC, D), jnp.float32) for _ in range(2)],
        [pltpu.VMEM((TD, TR, TC), jnp.float32) for _ in range(2)],
        pltpu.VMEM((L, D), jnp.float32),
        [pltpu.SemaphoreType.DMA for _ in range(2)],
        [pltpu.SemaphoreType.DMA for _ in range(2)],
        [pltpu.SemaphoreType.DMA for _ in range(2)],
    ],
)
def _sc_embed(idxT_hbm, tab_hbm, pos_hbm, out5, dummy, idx_vs, rows_vs,
              trans_vs, pos_v, isems, gsems, ssems):
    wid = lax.axis_index("s") * NC + lax.axis_index("c")
    b0 = wid * TC
    pltpu.sync_copy(pos_hbm, pos_v)
    iota = lax.iota(jnp.int32, 16)

    def issue_idx(u, sl):
        pltpu.async_copy(idxT_hbm.at[pl.ds(u, 1), pl.ds(b0, TC)],
                         idx_vs[sl], isems[sl])

    def issue_gather(sl):
        pltpu.async_copy(tab_hbm.at[idx_vs[sl].at[0]], rows_vs[sl], gsems[sl])

    def wait_idx(sl):
        pltpu.make_async_copy(idxT_hbm.at[pl.ds(0, 1), pl.ds(0, TC)],
                              idx_vs[sl], isems[sl]).wait()

    def wait_gather(sl):
        pltpu.make_async_copy(tab_hbm.at[pl.ds(0, TC)],
                              rows_vs[sl], gsems[sl]).wait()

    def wait_store(sl):
        pltpu.make_async_copy(dummy, trans_vs[sl], ssems[sl]).wait()

    GRP = 8
    tdv0 = iota >> 3
    tdv1 = tdv0 + 2
    rv = iota & 7

    def unit(l, sl):
        rows_v = rows_vs[sl]
        trans_v = trans_vs[sl]
        p0 = pos_v[l, pl.ds(0, 16)]
        p1 = pos_v[l, pl.ds(16, 16)]

        for c0 in range(0, TC, GRP):
            vals = []
            for c in range(c0, c0 + GRP):
                vals.append((rows_v[c, pl.ds(0, 16)] + p0,
                             rows_v[c, pl.ds(16, 16)] + p1))
            for c, (v0, v1) in zip(range(c0, c0 + GRP), vals):
                cv = jnp.full((16,), c, jnp.int32)
                plsc.store_scatter(trans_v, [tdv0, rv, cv], v0)
                plsc.store_scatter(trans_v, [tdv1, rv, cv], v1)
        pltpu.async_copy(trans_v, out5.at[l, :, wid], ssems[sl])

    pltpu.sync_copy(idxT_hbm.at[pl.ds(0, 1), pl.ds(b0, TC)], idx_vs[0])
    issue_gather(0)
    issue_idx(1, 1)

    def body(j, carry):
        for p in range(2):
            u = 2 * j + p
            q = 1 - p
            @pl.when(u + 1 < NU)
            def _():
                wait_idx(q)
                issue_gather(q)

            @pl.when(u + 2 < NU)
            def _():
                issue_idx(u + 2, p)

            @pl.when(u >= 2)
            def _():
                wait_store(p)

            wait_gather(p)
            unit(u, p)
        return carry

    lax.fori_loop(0, NU // 2, body, 0)
    wait_store(0)
    wait_store(1)


V = 1000000
TW = 16384
TG = -(-V // TW)


def _tc_relayout_body(x_ref, o_ref):
    y = x_ref[...].T.reshape(TW // 4, 4, D)
    o_ref[...] = jnp.concatenate([y[:, s, :] for s in range(4)], axis=-1)


_tc_relayout = pl.pallas_call(
    _tc_relayout_body,
    grid=(TG,),
    in_specs=[pl.BlockSpec((D, TW), lambda i: (0, i))],
    out_specs=pl.BlockSpec((TW // 4, 4 * D), lambda i: (i, 0)),
    out_shape=jax.ShapeDtypeStruct((V // 4, 4 * D), jnp.float32),
)


def kernel(inputs, token_table, pos_table):
    tab_lin = _tc_relayout(token_table.T).reshape(V, D)
    out5, _ = _sc_embed(inputs.T, tab_lin, pos_table)
    return out5.transpose(2, 4, 0, 1, 3).reshape(B, L, D)


# speedup vs baseline: 1.0157x; 1.0001x over previous
import functools

import jax
import jax.numpy as jnp
from jax import lax
from jax.experimental import pallas as pl
from jax.experimental.pallas import tpu as pltpu
from jax.experimental.pallas import tpu_sc as plsc

NC, NS = 2, 16
NW = NC * NS

B, L, D = 4096, 200, 32
TD, TR, TC = D // 8, 8, 128
NU = L


@functools.partial(
    pl.kernel,
    out_type=(
        jax.ShapeDtypeStruct((L, TD, NW, TR, TC), jnp.float32),
        jax.ShapeDtypeStruct((TD, TR, TC), jnp.float32),
    ),
    mesh=plsc.VectorSubcoreMesh(core_axis_name="c", subcore_axis_name="s"),
    compiler_params=pltpu.CompilerParams(
        use_tc_tiling_on_sc=False, needs_layout_passes=False),
    scratch_types=[
        [pltpu.VMEM((1, TC), jnp.int32) for _ in range(2)],
        [pltpu.VMEM((TC, D), jnp.float32) for _ in range(2)],
        [pltpu.VMEM((TD, TR, TC), jnp.float32) for _ in range(2)],
        pltpu.VMEM((L, D), jnp.float32),
        [pltpu.SemaphoreType.DMA for _ in range(2)],
        [pltpu.SemaphoreType.DMA for _ in range(2)],
        [pltpu.SemaphoreType.DMA for _ in range(2)],
    ],
)
def _sc_embed(idxT_hbm, tab_hbm, pos_hbm, out5, dummy, idx_vs, rows_vs,
              trans_vs, pos_v, isems, gsems, ssems):
    wid = lax.axis_index("s") * NC + lax.axis_index("c")
    b0 = wid * TC
    pltpu.sync_copy(pos_hbm, pos_v)
    iota = lax.iota(jnp.int32, 16)

    def issue_idx(u, sl):
        pltpu.async_copy(idxT_hbm.at[pl.ds(u, 1), pl.ds(b0, TC)],
                         idx_vs[sl], isems[sl])

    def issue_gather(sl):
        pltpu.async_copy(tab_hbm.at[idx_vs[sl].at[0, pl.ds(0, 64)]],
                         rows_vs[sl].at[pl.ds(0, 64)], gsems[sl])
        pltpu.async_copy(tab_hbm.at[idx_vs[sl].at[0, pl.ds(64, 64)]],
                         rows_vs[sl].at[pl.ds(64, 64)], gsems[sl])

    def wait_idx(sl):
        pltpu.make_async_copy(idxT_hbm.at[pl.ds(0, 1), pl.ds(0, TC)],
                              idx_vs[sl], isems[sl]).wait()

    def wait_gather(sl):
        pltpu.make_async_copy(tab_hbm.at[pl.ds(0, TC)],
                              rows_vs[sl], gsems[sl]).wait()

    def wait_store(sl):
        pltpu.make_async_copy(dummy, trans_vs[sl], ssems[sl]).wait()

    GRP = 8
    tdv0 = iota >> 3
    tdv1 = tdv0 + 2
    rv = iota & 7

    def unit(l, sl):
        rows_v = rows_vs[sl]
        trans_v = trans_vs[sl]
        p0 = pos_v[l, pl.ds(0, 16)]
        p1 = pos_v[l, pl.ds(16, 16)]

        for c0 in range(0, TC, GRP):
            vals = []
            for c in range(c0, c0 + GRP):
                vals.append((rows_v[c, pl.ds(0, 16)] + p0,
                             rows_v[c, pl.ds(16, 16)] + p1))
            for c, (v0, v1) in zip(range(c0, c0 + GRP), vals):
                cv = jnp.full((16,), c, jnp.int32)
                plsc.store_scatter(trans_v, [tdv0, rv, cv], v0)
                plsc.store_scatter(trans_v, [tdv1, rv, cv], v1)
        pltpu.async_copy(trans_v, out5.at[l, :, wid], ssems[sl])

    pltpu.sync_copy(idxT_hbm.at[pl.ds(0, 1), pl.ds(b0, TC)], idx_vs[0])
    issue_gather(0)
    issue_idx(1, 1)

    def body(j, carry):
        for p in range(2):
            u = 2 * j + p
            q = 1 - p
            @pl.when(u + 1 < NU)
            def _():
                wait_idx(q)
                issue_gather(q)

            @pl.when(u + 2 < NU)
            def _():
                issue_idx(u + 2, p)

            @pl.when(u >= 2)
            def _():
                wait_store(p)

            wait_gather(p)
            unit(u, p)
        return carry

    lax.fori_loop(0, NU // 2, body, 0)
    wait_store(0)
    wait_store(1)


V = 1000000
TW = 16384
TG = -(-V // TW)


def _tc_relayout_body(x_ref, o_ref):
    y = x_ref[...].T.reshape(TW // 4, 4, D)
    o_ref[...] = jnp.concatenate([y[:, s, :] for s in range(4)], axis=-1)


_tc_relayout = pl.pallas_call(
    _tc_relayout_body,
    grid=(TG,),
    in_specs=[pl.BlockSpec((D, TW), lambda i: (0, i))],
    out_specs=pl.BlockSpec((TW // 4, 4 * D), lambda i: (i, 0)),
    out_shape=jax.ShapeDtypeStruct((V // 4, 4 * D), jnp.float32),
)


def kernel(inputs, token_table, pos_table):
    tab_lin = _tc_relayout(token_table.T).reshape(V, D)
    out5, _ = _sc_embed(inputs.T, tab_lin, pos_table)
    return out5.transpose(2, 4, 0, 1, 3).reshape(B, L, D)


# speedup vs baseline: 1.0181x; 1.0023x over previous
import functools

import jax
import jax.numpy as jnp
from jax import lax
from jax.experimental import pallas as pl
from jax.experimental.pallas import tpu as pltpu
from jax.experimental.pallas import tpu_sc as plsc

NC, NS = 2, 16
NW = NC * NS

B, L, D = 4096, 200, 32
TD, TR, TC = D // 8, 8, 128
NU = L


@functools.partial(
    pl.kernel,
    out_type=(
        jax.ShapeDtypeStruct((L, TD, NW, TR, TC), jnp.float32),
        jax.ShapeDtypeStruct((TD, TR, TC), jnp.float32),
    ),
    mesh=plsc.VectorSubcoreMesh(core_axis_name="c", subcore_axis_name="s"),
    compiler_params=pltpu.CompilerParams(
        use_tc_tiling_on_sc=False, needs_layout_passes=False),
    scratch_types=[
        [pltpu.VMEM((1, TC), jnp.int32) for _ in range(2)],
        [pltpu.VMEM((TC, D), jnp.float32) for _ in range(2)],
        [pltpu.VMEM((TD, TR, TC), jnp.float32) for _ in range(2)],
        pltpu.VMEM((L, D), jnp.float32),
        [pltpu.SemaphoreType.DMA for _ in range(2)],
        [pltpu.SemaphoreType.DMA for _ in range(2)],
        [pltpu.SemaphoreType.DMA for _ in range(2)],
    ],
)
def _sc_embed(idxT_hbm, tab_hbm, pos_hbm, out5, dummy, idx_vs, rows_vs,
              trans_vs, pos_v, isems, gsems, ssems):
    wid = lax.axis_index("s") * NC + lax.axis_index("c")
    b0 = wid * TC
    pltpu.sync_copy(pos_hbm, pos_v)
    iota = lax.iota(jnp.int32, 16)

    def issue_idx(u, sl):
        pltpu.async_copy(idxT_hbm.at[pl.ds(u, 1), pl.ds(b0, TC)],
                         idx_vs[sl], isems[sl])

    def issue_gather(sl):
        pltpu.async_copy(tab_hbm.at[idx_vs[sl].at[0]], rows_vs[sl], gsems[sl])

    def wait_idx(sl):
        pltpu.make_async_copy(idxT_hbm.at[pl.ds(0, 1), pl.ds(0, TC)],
                              idx_vs[sl], isems[sl]).wait()

    def wait_gather(sl):
        pltpu.make_async_copy(tab_hbm.at[pl.ds(0, TC)],
                              rows_vs[sl], gsems[sl]).wait()

    def wait_store(sl):
        pltpu.make_async_copy(dummy, trans_vs[sl], ssems[sl]).wait()

    GRP = 8
    tdv0 = iota >> 3
    tdv1 = tdv0 + 2
    rv = iota & 7

    def unit(l, sl):
        rows_v = rows_vs[sl]
        trans_v = trans_vs[sl]
        p0 = pos_v[l, pl.ds(0, 16)]
        p1 = pos_v[l, pl.ds(16, 16)]

        for c0 in range(0, TC, GRP):
            vals = []
            for c in range(c0, c0 + GRP):
                vals.append((rows_v[c, pl.ds(0, 16)] + p0,
                             rows_v[c, pl.ds(16, 16)] + p1))
            for c, (v0, v1) in zip(range(c0, c0 + GRP), vals):
                cv = jnp.full((16,), c, jnp.int32)
                plsc.store_scatter(trans_v, [tdv0, rv, cv], v0)
                plsc.store_scatter(trans_v, [tdv1, rv, cv], v1)
        pltpu.async_copy(trans_v, out5.at[l, :, wid], ssems[sl])

    pltpu.sync_copy(idxT_hbm.at[pl.ds(0, 1), pl.ds(b0, TC)], idx_vs[0])
    issue_gather(0)
    issue_idx(1, 1)

    def body(j, carry):
        for p in range(2):
            u = 2 * j + p
            q = 1 - p
            @pl.when(u + 1 < NU)
            def _():
                wait_idx(q)
                issue_gather(q)

            @pl.when(u + 2 < NU)
            def _():
                issue_idx(u + 2, p)

            @pl.when(u >= 2)
            def _():
                wait_store(p)

            wait_gather(p)
            unit(u, p)
        return carry

    lax.fori_loop(0, NU // 2, body, 0)
    wait_store(0)
    wait_store(1)


V = 1000000
TW = 16384
TG = -(-V // TW)


def _tc_relayout_body(x_ref, o_ref):
    y = x_ref[...].T.reshape(TW // 4, 4, D)
    o_ref[...] = jnp.concatenate([y[:, s, :] for s in range(4)], axis=-1)


_tc_relayout = pl.pallas_call(
    _tc_relayout_body,
    grid=(TG,),
    in_specs=[pl.BlockSpec((D, TW), lambda i: (0, i))],
    out_specs=pl.BlockSpec((TW // 4, 4 * D), lambda i: (i, 0)),
    out_shape=jax.ShapeDtypeStruct((V // 4, 4 * D), jnp.float32),
)


def kernel(inputs, token_table, pos_table):
    tab_lin = _tc_relayout(token_table.T).reshape(V, D)
    out5, _ = _sc_embed(inputs.T, tab_lin, pos_table)
    return out5.transpose(2, 4, 0, 1, 3).reshape(B, L, D)
